# Initial kernel scaffold; baseline (speedup 1.0000x reference)
import jax
import jax.numpy as jnp
from jax.experimental import pallas as pl


def kernel(input, target):
    raise NotImplementedError("write your pallas kernel here")


# speedup vs baseline: 3.0333x; 3.0333x over previous
import math

import jax
import jax.numpy as jnp
from jax.experimental import pallas as pl
from jax.experimental.pallas import tpu as pltpu

B = 64
N = 100000
D = 1.5
NEG_C = -math.log(0.9)
CONST = 0.9 / math.exp(D)
BK = 2048
NBLK = (N + BK - 1) // BK
SENT = -1e30


def _body(x_ref, t_ref, out_ref, r1, r2, axt, at):
    pid = pl.program_id(0)

    @pl.when(pid == 0)
    def _init():
        r1[...] = jnp.full((B, 128), SENT, jnp.float32)
        r2[...] = jnp.full((B, 128), SENT, jnp.float32)
        axt[...] = jnp.zeros((B, 128), jnp.float32)
        at[...] = jnp.zeros((B, 128), jnp.float32)

    x = x_ref[...]
    t = t_ref[...]
    lane = jax.lax.broadcasted_iota(jnp.int32, (B, 128), 1)

    t1 = r1[...]
    t2 = r2[...]
    sxt = axt[...]
    st = at[...]
    for s in range(BK // 128):
        gcol = pid * BK + s * 128 + lane
        valid = gcol < N
        xs = x[:, s * 128:(s + 1) * 128]
        ts = t[:, s * 128:(s + 1) * 128]
        xm = jnp.where(valid & (ts <= 0.0), xs, SENT)
        t2 = jnp.maximum(t2, jnp.minimum(t1, xm))
        t1 = jnp.maximum(t1, xm)
        sxt = sxt + jnp.where(valid, xs * ts, 0.0)
        st = st + jnp.where(valid, ts, 0.0)
    r1[...] = t1
    r2[...] = t2
    axt[...] = sxt
    at[...] = st

    @pl.when(pid == NBLK - 1)
    def _fin():
        l1 = r1[...]
        l2 = r2[...]
        m1 = jnp.max(l1, axis=1, keepdims=True)
        idx1 = jnp.min(
            jnp.where(l1 == m1, lane, 1 << 20), axis=1, keepdims=True)
        m2 = jnp.max(jnp.where(lane == idx1, l2, l1), axis=1, keepdims=True)

        def neg_loss(v):
            return -jnp.log(1.0 - jnp.exp(D * v) * CONST)

        negs = jnp.sum(neg_loss(m1) + neg_loss(m2))
        p = jnp.sum(at[...])
        sx = jnp.sum(axt[...])
        total = p * (NEG_C + D) - D * sx + negs
        out_ref[0, 0] = total / (B * N)


def kernel(input, target):
    out = pl.pallas_call(
        _body,
        grid=(NBLK,),
        in_specs=[
            pl.BlockSpec((B, BK), lambda i: (0, i)),
            pl.BlockSpec((B, BK), lambda i: (0, i)),
        ],
        out_specs=pl.BlockSpec((1, 1), lambda i: (0, 0), memory_space=pltpu.SMEM),
        out_shape=jax.ShapeDtypeStruct((1, 1), jnp.float32),
        scratch_shapes=[pltpu.VMEM((B, 128), jnp.float32)] * 4,
    )(input, target)
    return jnp.reshape(out, ())
